# baseline (speedup 1.0000x reference)
import functools

import jax
import jax.numpy as jnp
from jax import lax
from jax.experimental import pallas as pl
from jax.experimental.pallas import tpu as pltpu
from jax.experimental.pallas import tpu_sc as plsc

_NUM_CORES = 2
_NUM_SUBCORES = 16
_NUM_WORKERS = _NUM_CORES * _NUM_SUBCORES
_LANES = 16
_CHUNK = 64


def _emb_lookup(ids_flat, word_embeddings, position_embeddings, *, batch,
                seqlen):
    _, d = word_embeddings.shape
    n = batch * seqlen
    s_per_w = seqlen // _NUM_WORKERS
    n_chunks = s_per_w // _CHUNK
    vecs_per_row = d // _LANES

    mesh = plsc.VectorSubcoreMesh(core_axis_name="c", subcore_axis_name="s")

    @functools.partial(
        pl.kernel,
        out_type=jax.ShapeDtypeStruct((n, d), jnp.float32),
        mesh=mesh,
        scratch_types=[
            pltpu.VMEM((_CHUNK,), jnp.int32),
            pltpu.VMEM((_CHUNK, d), jnp.float32),
            pltpu.VMEM((_CHUNK, d), jnp.float32),
            pltpu.SemaphoreType.DMA,
        ],
    )
    def body(ids_hbm, wtab_hbm, ptab_hbm, out_hbm, idx_v, rows_v, pos_v, sem):
        wid = lax.axis_index("s") * _NUM_CORES + lax.axis_index("c")
        s_base_w = wid * s_per_w

        def chunk_body(c, _):
            s0 = s_base_w + c * _CHUNK
            pltpu.sync_copy(ptab_hbm.at[pl.ds(s0, _CHUNK)], pos_v)
            for b in range(batch):
                base = b * seqlen + s0
                pltpu.sync_copy(ids_hbm.at[pl.ds(base, _CHUNK)], idx_v)
                pltpu.async_copy(wtab_hbm.at[idx_v], rows_v, sem).wait()

                def add_row(i, _):
                    def add_vec(j, _):
                        sl = pl.ds(j * _LANES, _LANES)
                        plsc.addupdate(rows_v.at[i, sl], pos_v[i, sl])
                        return ()
                    lax.fori_loop(0, vecs_per_row, add_vec, ())
                    return ()

                lax.fori_loop(0, _CHUNK, add_row, ())
                pltpu.sync_copy(rows_v, out_hbm.at[pl.ds(base, _CHUNK)])
            return ()

        lax.fori_loop(0, n_chunks, chunk_body, ())

    return body(ids_flat, word_embeddings, position_embeddings)


def kernel(input_ids, word_embeddings, position_embeddings):
    batch, seqlen = input_ids.shape
    _, d = word_embeddings.shape
    ids_flat = input_ids.reshape(batch * seqlen).astype(jnp.int32)
    out = _emb_lookup(
        ids_flat, word_embeddings, position_embeddings,
        batch=batch, seqlen=seqlen,
    )
    return out.reshape(batch, seqlen, d)


# speedup vs baseline: 1.9853x; 1.9853x over previous
import functools

import jax
import jax.numpy as jnp
from jax import lax
from jax.experimental import pallas as pl
from jax.experimental.pallas import tpu as pltpu
from jax.experimental.pallas import tpu_sc as plsc

_NUM_CORES = 2
_NUM_SUBCORES = 16
_NUM_WORKERS = _NUM_CORES * _NUM_SUBCORES
_LANES = 16
_CHUNK = 64


def _emb_lookup(ids_flat, word_embeddings, position_embeddings, *, batch,
                seqlen):
    _, d = word_embeddings.shape
    n = batch * seqlen
    s_per_w = seqlen // _NUM_WORKERS
    n_chunks = s_per_w // _CHUNK
    vecs_per_row = d // _LANES

    mesh = plsc.VectorSubcoreMesh(core_axis_name="c", subcore_axis_name="s")

    @functools.partial(
        pl.kernel,
        out_type=jax.ShapeDtypeStruct((n, d), jnp.float32),
        mesh=mesh,
        scratch_types=[
            pltpu.VMEM((_CHUNK,), jnp.int32),
            pltpu.VMEM((_CHUNK, d), jnp.float32),
            pltpu.VMEM((_CHUNK, d), jnp.float32),
            pltpu.SemaphoreType.DMA,
        ],
    )
    def body(ids_hbm, wtab_hbm, ptab_hbm, out_hbm, idx_v, rows_v, pos_v, sem):
        wid = lax.axis_index("s") * _NUM_CORES + lax.axis_index("c")
        s_base_w = wid * s_per_w

        def chunk_body(c, _):
            s0 = s_base_w + c * _CHUNK
            pltpu.sync_copy(ptab_hbm.at[pl.ds(s0, _CHUNK)], pos_v)
            for b in range(batch):
                base = b * seqlen + s0
                pltpu.sync_copy(ids_hbm.at[pl.ds(base, _CHUNK)], idx_v)
                pltpu.async_copy(wtab_hbm.at[idx_v], rows_v, sem).wait()

                def add_row(i, _):
                    for j in range(vecs_per_row):
                        sl = pl.ds(j * _LANES, _LANES)
                        plsc.addupdate(rows_v.at[i, sl], pos_v[i, sl])
                    return ()

                lax.fori_loop(0, _CHUNK, add_row, ())
                pltpu.sync_copy(rows_v, out_hbm.at[pl.ds(base, _CHUNK)])
            return ()

        lax.fori_loop(0, n_chunks, chunk_body, ())

    return body(ids_flat, word_embeddings, position_embeddings)


def kernel(input_ids, word_embeddings, position_embeddings):
    batch, seqlen = input_ids.shape
    _, d = word_embeddings.shape
    ids_flat = input_ids.reshape(batch * seqlen).astype(jnp.int32)
    out = _emb_lookup(
        ids_flat, word_embeddings, position_embeddings,
        batch=batch, seqlen=seqlen,
    )
    return out.reshape(batch, seqlen, d)
